# baseline (device time: 41498 ns/iter reference)
import jax
import jax.numpy as jnp
from jax import lax
from jax.experimental import pallas as pl
from jax.experimental.pallas import tpu as pltpu

T = 512
D = 512
F = 1024
E_LOC = 2


def kernel(x, assign, W1, W2):
    assign2d = assign.reshape(T, 1)

    def body(x_ref, a_ref, w1_ref, w2_ref, out_ref,
             xrecv, arecv, accrem, partner, send_sems, recv_sems):
        my_x = lax.axis_index("x")
        my_y = lax.axis_index("y")
        nbr = (my_x, 1 - my_y)

        barrier_sem = pltpu.get_barrier_semaphore()
        pl.semaphore_signal(barrier_sem, inc=1, device_id=nbr,
                            device_id_type=pl.DeviceIdType.MESH)
        pl.semaphore_wait(barrier_sem, 1)

        rdma_x = pltpu.make_async_remote_copy(
            src_ref=x_ref, dst_ref=xrecv,
            send_sem=send_sems.at[0], recv_sem=recv_sems.at[0],
            device_id=nbr, device_id_type=pl.DeviceIdType.MESH,
        )
        rdma_a = pltpu.make_async_remote_copy(
            src_ref=a_ref, dst_ref=arecv,
            send_sem=send_sems.at[1], recv_sem=recv_sems.at[1],
            device_id=nbr, device_id_type=pl.DeviceIdType.MESH,
        )
        rdma_x.start()
        rdma_a.start()

        def moe(tokens, assigns):
            acc = jnp.zeros((T, D), jnp.float32)
            for k in range(E_LOC):
                e = E_LOC * my_y + k
                h = jnp.maximum(
                    jnp.dot(tokens, w1_ref[k], preferred_element_type=jnp.float32),
                    0.0)
                y = jnp.dot(h, w2_ref[k], preferred_element_type=jnp.float32)
                acc = acc + jnp.where(assigns == e, y, 0.0)
            return acc

        acc_local = moe(x_ref[:, :], a_ref[:, :])

        rdma_x.wait()
        rdma_a.wait()

        accrem[:, :] = moe(xrecv[:, :], arecv[:, :])

        rdma_b = pltpu.make_async_remote_copy(
            src_ref=accrem, dst_ref=partner,
            send_sem=send_sems.at[2], recv_sem=recv_sems.at[2],
            device_id=nbr, device_id_type=pl.DeviceIdType.MESH,
        )
        rdma_b.start()
        rdma_b.wait()

        out_ref[:, :] = acc_local + partner[:, :]

    return pl.pallas_call(
        body,
        out_shape=jax.ShapeDtypeStruct((T, D), jnp.float32),
        in_specs=[
            pl.BlockSpec(memory_space=pltpu.VMEM),
            pl.BlockSpec(memory_space=pltpu.VMEM),
            pl.BlockSpec(memory_space=pltpu.VMEM),
            pl.BlockSpec(memory_space=pltpu.VMEM),
        ],
        out_specs=pl.BlockSpec(memory_space=pltpu.VMEM),
        scratch_shapes=[
            pltpu.VMEM((T, D), jnp.float32),
            pltpu.VMEM((T, 1), jnp.int32),
            pltpu.VMEM((T, D), jnp.float32),
            pltpu.VMEM((T, D), jnp.float32),
            pltpu.SemaphoreType.DMA((3,)),
            pltpu.SemaphoreType.DMA((3,)),
        ],
        compiler_params=pltpu.CompilerParams(collective_id=0),
    )(x, assign2d, W1, W2)


# device time: 37960 ns/iter; 1.0932x vs baseline; 1.0932x over previous
import jax
import jax.numpy as jnp
from jax import lax
from jax.experimental import pallas as pl
from jax.experimental.pallas import tpu as pltpu

T = 512
D = 512
F = 1024
E_LOC = 2
NCHUNK = 4


def kernel(x, assign, W1, W2):
    assign2d = assign.reshape(T, 1)

    def body(x_ref, a_ref, w1_ref, w2_ref, out_ref,
             xrecv, arecv, accrem, partner,
             send_sems, recv_sems, ret_send_sems, ret_recv_sems):
        my_x = lax.axis_index("x")
        my_y = lax.axis_index("y")
        nbr = (my_x, 1 - my_y)

        barrier_sem = pltpu.get_barrier_semaphore()
        pl.semaphore_signal(barrier_sem, inc=1, device_id=nbr,
                            device_id_type=pl.DeviceIdType.MESH)
        pl.semaphore_wait(barrier_sem, 1)

        rdma_x = pltpu.make_async_remote_copy(
            src_ref=x_ref, dst_ref=xrecv,
            send_sem=send_sems.at[0], recv_sem=recv_sems.at[0],
            device_id=nbr, device_id_type=pl.DeviceIdType.MESH,
        )
        rdma_a = pltpu.make_async_remote_copy(
            src_ref=a_ref, dst_ref=arecv,
            send_sem=send_sems.at[1], recv_sem=recv_sems.at[1],
            device_id=nbr, device_id_type=pl.DeviceIdType.MESH,
        )
        rdma_x.start()
        rdma_a.start()

        def moe(tokens, assigns):
            acc = jnp.zeros((tokens.shape[0], D), jnp.float32)
            for k in range(E_LOC):
                e = E_LOC * my_y + k
                h = jnp.maximum(
                    jnp.dot(tokens, w1_ref[k], preferred_element_type=jnp.float32),
                    0.0)
                y = jnp.dot(h, w2_ref[k], preferred_element_type=jnp.float32)
                acc = acc + jnp.where(assigns == e, y, 0.0)
            return acc

        acc_local = moe(x_ref[:, :], a_ref[:, :])

        rdma_x.wait()
        rdma_a.wait()

        CH = T // NCHUNK
        rets = []
        for c in range(NCHUNK):
            rows = slice(c * CH, (c + 1) * CH)
            accrem[rows, :] = moe(xrecv[rows, :], arecv[rows, :])
            r = pltpu.make_async_remote_copy(
                src_ref=accrem.at[rows],
                dst_ref=partner.at[rows],
                send_sem=ret_send_sems.at[c], recv_sem=ret_recv_sems.at[c],
                device_id=nbr, device_id_type=pl.DeviceIdType.MESH,
            )
            r.start()
            rets.append(r)

        for c, r in enumerate(rets):
            rows = slice(c * CH, (c + 1) * CH)
            r.wait_recv()
            out_ref[rows, :] = acc_local[rows, :] + partner[rows, :]
        for r in rets:
            r.wait_send()

    return pl.pallas_call(
        body,
        out_shape=jax.ShapeDtypeStruct((T, D), jnp.float32),
        in_specs=[
            pl.BlockSpec(memory_space=pltpu.VMEM),
            pl.BlockSpec(memory_space=pltpu.VMEM),
            pl.BlockSpec(memory_space=pltpu.VMEM),
            pl.BlockSpec(memory_space=pltpu.VMEM),
        ],
        out_specs=pl.BlockSpec(memory_space=pltpu.VMEM),
        scratch_shapes=[
            pltpu.VMEM((T, D), jnp.float32),
            pltpu.VMEM((T, 1), jnp.int32),
            pltpu.VMEM((T, D), jnp.float32),
            pltpu.VMEM((T, D), jnp.float32),
            pltpu.SemaphoreType.DMA((2,)),
            pltpu.SemaphoreType.DMA((2,)),
            pltpu.SemaphoreType.DMA((NCHUNK,)),
            pltpu.SemaphoreType.DMA((NCHUNK,)),
        ],
        compiler_params=pltpu.CompilerParams(collective_id=0),
    )(x, assign2d, W1, W2)


# device time: 27461 ns/iter; 1.5112x vs baseline; 1.3823x over previous
import jax
import jax.numpy as jnp
from jax import lax
from jax.experimental import pallas as pl
from jax.experimental.pallas import tpu as pltpu

T = 512
D = 512
F = 1024
E_LOC = 2
NCHUNK = 4


def kernel(x, assign, W1, W2):
    assign2d = assign.reshape(T, 1)
    xb = x.astype(jnp.bfloat16)
    w1b = W1.astype(jnp.bfloat16)
    w2b = W2.astype(jnp.bfloat16)

    def body(x_ref, a_ref, w1_ref, w2_ref, out_ref,
             xrecv, arecv, accrem, partner,
             send_sems, recv_sems, ret_send_sems, ret_recv_sems):
        my_x = lax.axis_index("x")
        my_y = lax.axis_index("y")
        nbr = (my_x, 1 - my_y)

        barrier_sem = pltpu.get_barrier_semaphore()
        pl.semaphore_signal(barrier_sem, inc=1, device_id=nbr,
                            device_id_type=pl.DeviceIdType.MESH)
        pl.semaphore_wait(barrier_sem, 1)

        rdma_x = pltpu.make_async_remote_copy(
            src_ref=x_ref, dst_ref=xrecv,
            send_sem=send_sems.at[0], recv_sem=recv_sems.at[0],
            device_id=nbr, device_id_type=pl.DeviceIdType.MESH,
        )
        rdma_a = pltpu.make_async_remote_copy(
            src_ref=a_ref, dst_ref=arecv,
            send_sem=send_sems.at[1], recv_sem=recv_sems.at[1],
            device_id=nbr, device_id_type=pl.DeviceIdType.MESH,
        )
        rdma_x.start()
        rdma_a.start()

        def moe(tokens, assigns):
            acc = jnp.zeros((tokens.shape[0], D), jnp.float32)
            for k in range(E_LOC):
                e = E_LOC * my_y + k
                h = jnp.maximum(
                    jnp.dot(tokens, w1_ref[k], preferred_element_type=jnp.float32),
                    0.0).astype(jnp.bfloat16)
                y = jnp.dot(h, w2_ref[k], preferred_element_type=jnp.float32)
                acc = acc + jnp.where(assigns == e, y, 0.0)
            return acc

        acc_local = moe(x_ref[:, :], a_ref[:, :])

        rdma_x.wait()
        rdma_a.wait()

        CH = T // NCHUNK
        rets = []
        for c in range(NCHUNK):
            rows = slice(c * CH, (c + 1) * CH)
            accrem[rows, :] = moe(xrecv[rows, :], arecv[rows, :]).astype(jnp.bfloat16)
            r = pltpu.make_async_remote_copy(
                src_ref=accrem.at[rows],
                dst_ref=partner.at[rows],
                send_sem=ret_send_sems.at[c], recv_sem=ret_recv_sems.at[c],
                device_id=nbr, device_id_type=pl.DeviceIdType.MESH,
            )
            r.start()
            rets.append(r)

        for c, r in enumerate(rets):
            rows = slice(c * CH, (c + 1) * CH)
            r.wait_recv()
            out_ref[rows, :] = acc_local[rows, :] + partner[rows, :].astype(jnp.float32)
        for r in rets:
            r.wait_send()

    return pl.pallas_call(
        body,
        out_shape=jax.ShapeDtypeStruct((T, D), jnp.float32),
        in_specs=[
            pl.BlockSpec(memory_space=pltpu.VMEM),
            pl.BlockSpec(memory_space=pltpu.VMEM),
            pl.BlockSpec(memory_space=pltpu.VMEM),
            pl.BlockSpec(memory_space=pltpu.VMEM),
        ],
        out_specs=pl.BlockSpec(memory_space=pltpu.VMEM),
        scratch_shapes=[
            pltpu.VMEM((T, D), jnp.bfloat16),
            pltpu.VMEM((T, 1), jnp.int32),
            pltpu.VMEM((T, D), jnp.bfloat16),
            pltpu.VMEM((T, D), jnp.bfloat16),
            pltpu.SemaphoreType.DMA((2,)),
            pltpu.SemaphoreType.DMA((2,)),
            pltpu.SemaphoreType.DMA((NCHUNK,)),
            pltpu.SemaphoreType.DMA((NCHUNK,)),
        ],
        compiler_params=pltpu.CompilerParams(collective_id=0),
    )(xb, assign2d, w1b, w2b)
